# baseline (speedup 1.0000x reference)
import functools

import jax
import jax.numpy as jnp
from jax import lax
from jax.experimental import pallas as pl
from jax.experimental.pallas import tpu as pltpu
from jax.experimental.pallas import tpu_sc as plsc

_NUM_SUBCORES = 16
_B = 4096
_ROW = 128
_RPW = _B // _ROW // _NUM_SUBCORES

_mesh = plsc.VectorSubcoreMesh(
    core_axis_name="c", subcore_axis_name="s", num_cores=1, num_subcores=16
)


@functools.partial(
    pl.kernel,
    out_type=jax.ShapeDtypeStruct((_B // _ROW, _ROW), jnp.float32),
    mesh=_mesh,
    scratch_types=[
        pltpu.VMEM((_RPW, _ROW), jnp.int32),
        pltpu.VMEM((_RPW, _ROW), jnp.float32),
        pltpu.SemaphoreType.DMA,
    ],
    compiler_params=pltpu.CompilerParams(needs_layout_passes=False),
)
def _gather_sc(betas_hbm, t_hbm, out_hbm, idx_v, out_v, sem):
    wid = lax.axis_index("s")
    base = wid * _RPW
    pltpu.sync_copy(t_hbm.at[pl.ds(base, _RPW)], idx_v)
    copies = [
        pltpu.async_copy(betas_hbm.at[idx_v.at[j]], out_v.at[j], sem)
        for j in range(_RPW)
    ]
    for c in copies:
        c.wait()
    pltpu.sync_copy(out_v, out_hbm.at[pl.ds(base, _RPW)])


def kernel(betas, t_int):
    t2 = t_int.astype(jnp.int32).reshape(_B // _ROW, _ROW)
    out = _gather_sc(betas.astype(jnp.float32), t2)
    return out.reshape(_B)


# speedup vs baseline: 1.1117x; 1.1117x over previous
import functools

import jax
import jax.numpy as jnp
from jax import lax
from jax.experimental import pallas as pl
from jax.experimental.pallas import tpu as pltpu
from jax.experimental.pallas import tpu_sc as plsc

_LANES = 16
_NUM_CORES = 2
_NUM_SUBCORES = 16
_NW = 1 * _NUM_SUBCORES
_B = 4096
_BPW = _B // _NW
_TABLE = 1001

_mesh = plsc.VectorSubcoreMesh(
    core_axis_name="c", subcore_axis_name="s", num_cores=1, num_subcores=16
)


@functools.partial(
    pl.kernel,
    out_type=jax.ShapeDtypeStruct((_B,), jnp.float32),
    mesh=_mesh,
    scratch_types=[
        pltpu.VMEM((_TABLE,), jnp.float32),
        pltpu.VMEM((_BPW,), jnp.int32),
        pltpu.VMEM((_BPW,), jnp.float32),
        pltpu.SemaphoreType.DMA,
        pltpu.SemaphoreType.DMA,
    ],
    compiler_params=pltpu.CompilerParams(
        needs_layout_passes=False,
        disable_bounds_checks=True,
        disable_semaphore_checks=True,
        skip_device_barrier=True,
    ),
)
def _gather_sc(betas_hbm, t_hbm, out_hbm, table_v, idx_v, out_v, sem_t, sem_i):
    wid = lax.axis_index("s")
    base = wid * _BPW
    tbl_cp = pltpu.async_copy(betas_hbm, table_v, sem_t)
    idx_cp = pltpu.async_copy(t_hbm.at[pl.ds(base, _BPW)], idx_v, sem_i)
    idx_cp.wait()
    tbl_cp.wait()
    for j in range(_BPW // _LANES):
        idx = idx_v[pl.ds(j * _LANES, _LANES)]
        out_v[pl.ds(j * _LANES, _LANES)] = plsc.load_gather(table_v, [idx])
    pltpu.sync_copy(out_v, out_hbm.at[pl.ds(base, _BPW)])


def kernel(betas, t_int):
    return _gather_sc(betas.astype(jnp.float32), t_int.astype(jnp.int32))
